# Initial kernel scaffold; baseline (speedup 1.0000x reference)
import jax
import jax.numpy as jnp
from jax.experimental import pallas as pl


def kernel(x_main, x_mod, x_proj, csr_idx, W1, b1, bn_gamma, bn_beta, W2, b2, gate_w, gate_b):
    raise NotImplementedError("write your pallas kernel here")


# speedup vs baseline: 8.3316x; 8.3316x over previous
import functools

import jax
import jax.numpy as jnp
from jax import lax
from jax.experimental import pallas as pl
from jax.experimental.pallas import tpu as pltpu
from jax.experimental.pallas import tpu_sc as plsc

N = 10000
V = 320000
F_PROJ = 64
F_MOD = 128
G = 8
EPS_BN = 1e-5

NW = 32
SPW = 320
NPAD = NW * SPW
CSR_COPY = 328
CSR_BUF = 344
CR1 = 512
CR2 = 256
TCBLK = 2000


def _tc_h_stats_body(xp_ref, w1_ref, b1_ref, h_ref, stats_ref):
    i = pl.program_id(0)
    x = xp_ref[...]
    h = lax.dot_general(x, w1_ref[...], (((1,), (1,)), ((), ())),
                        preferred_element_type=jnp.float32) + b1_ref[...]
    h_ref[...] = h
    s1 = jnp.sum(h, axis=0, keepdims=True)
    s2 = jnp.sum(h * h, axis=0, keepdims=True)
    upd = jnp.concatenate([s1, s2], axis=0)

    @pl.when(i == 0)
    def _():
        stats_ref[...] = upd

    @pl.when(i > 0)
    def _():
        stats_ref[...] += upd


def _tc_compat_body(h_ref, a_ref, bb_ref, w2_ref, b2_ref, out_ref):
    hn = jnp.maximum(h_ref[...] * a_ref[...] + bb_ref[...], 0.0)
    out_ref[...] = lax.dot_general(hn, w2_ref[...], (((1,), (1,)), ((), ())),
                                   preferred_element_type=jnp.float32) + b2_ref[...]


def _mlp_h_stats(x_proj, w1p, b1p):
    nblk = V // TCBLK
    return pl.pallas_call(
        _tc_h_stats_body,
        grid=(nblk,),
        in_specs=[
            pl.BlockSpec((TCBLK, F_PROJ), lambda i: (i, 0)),
            pl.BlockSpec((8, F_PROJ), lambda i: (0, 0)),
            pl.BlockSpec((1, 8), lambda i: (0, 0)),
        ],
        out_specs=[
            pl.BlockSpec((TCBLK, 8), lambda i: (i, 0)),
            pl.BlockSpec((2, 8), lambda i: (0, 0)),
        ],
        out_shape=[
            jax.ShapeDtypeStruct((V, 8), jnp.float32),
            jax.ShapeDtypeStruct((2, 8), jnp.float32),
        ],
    )(x_proj, w1p, b1p)


def _mlp_compat(h, a, bb, w2p, b2p):
    nblk = V // TCBLK
    return pl.pallas_call(
        _tc_compat_body,
        grid=(nblk,),
        in_specs=[
            pl.BlockSpec((TCBLK, 8), lambda i: (i, 0)),
            pl.BlockSpec((1, 8), lambda i: (0, 0)),
            pl.BlockSpec((1, 8), lambda i: (0, 0)),
            pl.BlockSpec((8, 8), lambda i: (0, 0)),
            pl.BlockSpec((1, 8), lambda i: (0, 0)),
        ],
        out_specs=pl.BlockSpec((TCBLK, 8), lambda i: (i, 0)),
        out_shape=jax.ShapeDtypeStruct((V, 8), jnp.float32),
    )(h, a, bb, w2p, b2p)


def _rsqrt_scalar(x):
    i = lax.bitcast_convert_type(x, jnp.int32)
    i = jnp.int32(0x5F3759DF) - lax.shift_right_logical(i, 1)
    y = lax.bitcast_convert_type(i, jnp.float32)
    for _ in range(3):
        y = y * (1.5 - 0.5 * x * y * y)
    return y


_SPLAT_DNUMS = lax.GatherDimensionNumbers(
    offset_dims=(), collapsed_slice_dims=(0,), start_index_map=(0,))


def _splat(vec, lane):
    idx = jnp.full((16, 1), lane, jnp.int32)
    return lax.gather(vec, idx, dimension_numbers=_SPLAT_DNUMS,
                      slice_sizes=(1,),
                      mode=lax.GatherScatterMode.PROMISE_IN_BOUNDS)


def _sget(ref, idx):
    return ref[pl.ds(idx, 16)][0]


def _sc_body(csr_hbm, compat_hbm, xmod_hbm, gpar_hbm, out_hbm,
             csr_v, mbuf, outbuf, cbuf1, cbuf2, xbuf, gparv):
    wid = lax.axis_index("c") * 16 + lax.axis_index("s")
    _sc_worker(wid, csr_hbm, compat_hbm, xmod_hbm, gpar_hbm, out_hbm,
               csr_v, mbuf, outbuf, cbuf1, cbuf2, xbuf, gparv)


def _sc_worker(wid, csr_hbm, compat_hbm, xmod_hbm, gpar_hbm, out_hbm,
               csr_v, mbuf, outbuf, cbuf1, cbuf2, xbuf, gparv):
    base_seg = wid * SPW

    pltpu.sync_copy(csr_hbm.at[pl.ds(base_seg, CSR_COPY)],
                    csr_v.at[pl.ds(0, CSR_COPY)])
    pltpu.sync_copy(gpar_hbm, gparv)

    r0 = _sget(csr_v, 0)
    r1 = _sget(csr_v, SPW)
    lanes = lax.iota(jnp.int32, 16)
    mask8 = lanes < 8
    neg = jnp.float32(-jnp.inf)
    minf = jnp.full((16,), neg, jnp.float32)
    zero16 = jnp.zeros((16,), jnp.float32)
    zacc = (zero16,) * 8

    def binsearch(r):
        def step(_, lohi):
            lo, hi = lohi
            mid = lax.shift_right_logical(lo + hi, 1)
            go = _sget(csr_v, mid) <= r
            return (jnp.where(go, mid, lo), jnp.where(go, hi, mid))

        lo, _ = lax.fori_loop(0, 9, step, (jnp.int32(0), jnp.int32(SPW)))
        return lo

    nch1 = (r1 - r0 + (CR1 - 1)) // CR1

    def chunk1(k, carry):
        cursor, m_carry = carry
        off = r0 + k * CR1
        offc = jnp.minimum(off, V - CR1)
        sh = off - offc
        pltpu.sync_copy(compat_hbm.at[pl.ds(offc * 8, CR1 * 8)],
                        cbuf1.at[pl.ds(0, CR1 * 8)])
        nr = jnp.minimum(CR1, r1 - off)
        end = off + nr
        seg_hi = binsearch(end - 1)

        def piece(s2, m_cur):
            a = _sget(csr_v, s2)
            b = _sget(csr_v, s2 + 1)
            a2 = jnp.maximum(a, off)
            b2 = jnp.minimum(b, end)
            m_cur = jnp.where(a >= off, minf, m_cur)

            def rowloop(j, mc):
                cvec = cbuf1[pl.ds((j + sh) * 8, 16)]
                return jnp.maximum(mc, jnp.where(mask8, cvec, neg))

            m_cur = lax.fori_loop(a2 - off, b2 - off, rowloop, m_cur)

            @pl.when(b <= end)
            def _():
                mbuf[pl.ds(s2 * 16, 16)] = m_cur

            return m_cur

        m_carry = lax.fori_loop(cursor, seg_hi + 1, piece, m_carry)
        cursor = jnp.where(_sget(csr_v, seg_hi + 1) <= end, seg_hi + 1, seg_hi)
        return (cursor, m_carry)

    _ = lax.fori_loop(0, nch1, chunk1, (jnp.int32(0), minf))

    gw = gparv[pl.ds(0, 16)]
    gb = gparv[pl.ds(16, 16)]

    def finalize(sg, denom, accs, m_c):
        n = _sget(csr_v, sg + 1) - _sget(csr_v, sg)
        m0 = jnp.where(mask8, m_c, 0.0)
        z = jnp.maximum(m0 * gw + gb, 0.0)
        ez = jnp.exp(z + z)
        gate = 1.0 - 2.0 / (ez + 1.0)
        scale = gate / (denom + 1e-12)
        scale = jnp.where(n > 0, scale, 0.0)
        for v in range(8):
            outbuf[pl.ds(sg * 128 + v * 16, 16)] = accs[v] * _splat(scale, v)

    nch2 = (r1 - r0 + (CR2 - 1)) // CR2

    def chunk2(k, carry):
        cursor, denom_carry, accs_carry = carry
        off = r0 + k * CR2
        offc = jnp.minimum(off, V - CR2)
        sh = off - offc
        pltpu.sync_copy(compat_hbm.at[pl.ds(offc * 8, CR2 * 8)],
                        cbuf2.at[pl.ds(0, CR2 * 8)])
        pltpu.sync_copy(xmod_hbm.at[pl.ds(offc * 128, CR2 * 128)], xbuf)
        nr = jnp.minimum(CR2, r1 - off)
        end = off + nr
        seg_hi = binsearch(end - 1)

        def piece(s2, c2):
            denom, accs = c2
            a = _sget(csr_v, s2)
            b = _sget(csr_v, s2 + 1)
            a2 = jnp.maximum(a, off)
            b2 = jnp.minimum(b, end)
            fresh = a >= off
            denom = jnp.where(fresh, 0.0, denom)
            accs = tuple(jnp.where(fresh, 0.0, ac) for ac in accs)
            m_c = mbuf[pl.ds(s2 * 16, 16)]
            isn = _rsqrt_scalar(jnp.maximum(b - a, 1).astype(jnp.float32))

            def rowloop(j, c3):
                dn, ac = c3
                jb = j + sh
                cvec = cbuf2[pl.ds(jb * 8, 16)]
                e = jnp.exp((cvec - m_c) * isn)
                dn = dn + jnp.where(mask8, e, 0.0)
                ac = tuple(
                    ac[v] + _splat(e, v) * xbuf[pl.ds(jb * 128 + v * 16, 16)]
                    for v in range(8))
                return (dn, ac)

            denom, accs = lax.fori_loop(a2 - off, b2 - off, rowloop,
                                        (denom, accs))

            @pl.when(b <= end)
            def _():
                finalize(s2, denom, accs, m_c)

            return (denom, accs)

        denom_carry, accs_carry = lax.fori_loop(cursor, seg_hi + 1, piece,
                                                (denom_carry, accs_carry))
        cursor = jnp.where(_sget(csr_v, seg_hi + 1) <= end, seg_hi + 1, seg_hi)
        return (cursor, denom_carry, accs_carry)

    cursorf, _, _ = lax.fori_loop(0, nch2, chunk2,
                                  (jnp.int32(0), zero16, zacc))

    def tailseg(s2, _):
        finalize(s2, zero16, zacc, zero16)
        return 0

    _ = lax.fori_loop(cursorf, SPW, tailseg, 0)

    pltpu.sync_copy(outbuf, out_hbm.at[pl.ds(base_seg * 128, SPW * 128)])


def _build_sc_pool(interpret=False):
    mesh = plsc.VectorSubcoreMesh(core_axis_name="c", subcore_axis_name="s")
    return pl.kernel(
        _sc_body,
        out_type=jax.ShapeDtypeStruct((NPAD * 128,), jnp.float32),
        mesh=mesh,
        scratch_types=[
            pltpu.VMEM((CSR_BUF,), jnp.int32),
            pltpu.VMEM((SPW * 16,), jnp.float32),
            pltpu.VMEM((SPW * 128,), jnp.float32),
            pltpu.VMEM((CR1 * 8 + 16,), jnp.float32),
            pltpu.VMEM((CR2 * 8 + 16,), jnp.float32),
            pltpu.VMEM((CR2 * 128,), jnp.float32),
            pltpu.VMEM((32,), jnp.float32),
        ],
        interpret=interpret,
    )


_sc_pool = _build_sc_pool()


def kernel(x_main, x_mod, x_proj, csr_idx, W1, b1, bn_gamma, bn_beta,
           W2, b2, gate_w, gate_b):
    del x_main
    w1p = jnp.zeros((8, F_PROJ), jnp.float32).at[:4].set(W1)
    b1p = jnp.zeros((1, 8), jnp.float32).at[0, :4].set(b1)
    h, stats = _mlp_h_stats(x_proj, w1p, b1p)
    mu = stats[0] / V
    var = stats[1] / V - mu * mu
    gam = jnp.zeros((8,), jnp.float32).at[:4].set(bn_gamma)
    bet = jnp.zeros((8,), jnp.float32).at[:4].set(bn_beta)
    a = gam * lax.rsqrt(var + EPS_BN)
    bb = bet - mu * a
    w2p = jnp.zeros((8, 8), jnp.float32).at[:, :4].set(W2)
    compat = _mlp_compat(h, a.reshape(1, 8), bb.reshape(1, 8), w2p,
                         b2.reshape(1, 8))

    csr32 = csr_idx.astype(jnp.int32)
    csr_pad = jnp.concatenate(
        [csr32, jnp.full(((NW - 1) * SPW + CSR_COPY - (N + 1),), V, jnp.int32)])
    gpar = jnp.concatenate([
        jnp.tile(gate_w.reshape(-1), 2).astype(jnp.float32),
        jnp.tile(gate_b.reshape(-1), 2).astype(jnp.float32),
    ])
    out_flat = _sc_pool(csr_pad, compat.reshape(-1), x_mod.reshape(-1), gpar)
    x_pool = out_flat.reshape(NPAD, F_MOD)[:N]
    x_seen = csr_idx[1:] > csr_idx[:-1]
    return x_pool, x_seen


# speedup vs baseline: 12.1427x; 1.4574x over previous
import jax
import jax.numpy as jnp
from jax import lax
from jax.experimental import pallas as pl
from jax.experimental.pallas import tpu as pltpu
from jax.experimental.pallas import tpu_sc as plsc

N = 10000
V = 320000
F_PROJ = 64
F_MOD = 128
G = 8
EPS_BN = 1e-5

NW = 32
SPW = 320
NPAD = NW * SPW
CSR_COPY = 328
CSR_BUF = 344
CR1 = 512
CR2 = 256
TCBLK = 2560
TCBLKB = 12800


def _tc_h_stats_body(xp_ref, w1_ref, ht_ref, stats_ref):
    i = pl.program_id(0)
    x = xp_ref[...]
    ht = lax.dot_general(w1_ref[...], x, (((1,), (1,)), ((), ())),
                         preferred_element_type=jnp.float32)
    ht_ref[...] = ht
    s1 = jnp.sum(ht, axis=1, keepdims=True)
    s2 = jnp.sum(ht * ht, axis=1, keepdims=True)
    upd = jnp.pad(jnp.concatenate([s1, s2], axis=1), ((0, 0), (0, 126)))

    @pl.when(i == 0)
    def _():
        stats_ref[...] = upd

    @pl.when(i > 0)
    def _():
        stats_ref[...] += upd


def _tc_compat_body(ht_ref, a_ref, bb_ref, w2_ref, b2_ref, out_ref):
    a = a_ref[...][:, 0:1]
    bb = bb_ref[...][:, 0:1]
    b2 = b2_ref[...][:, 0:1]
    hn = jnp.maximum(ht_ref[...] * a + bb, 0.0)
    out_ref[...] = lax.dot_general(w2_ref[...], hn, (((1,), (0,)), ((), ())),
                                   preferred_element_type=jnp.float32) + b2


def _mlp_h_stats(x_proj, w1p):
    nblk = V // TCBLK
    return pl.pallas_call(
        _tc_h_stats_body,
        grid=(nblk,),
        in_specs=[
            pl.BlockSpec((TCBLK, F_PROJ), lambda i: (i, 0)),
            pl.BlockSpec((8, F_PROJ), lambda i: (0, 0)),
        ],
        out_specs=[
            pl.BlockSpec((8, TCBLK), lambda i: (0, i)),
            pl.BlockSpec((8, 128), lambda i: (0, 0)),
        ],
        out_shape=[
            jax.ShapeDtypeStruct((8, V), jnp.float32),
            jax.ShapeDtypeStruct((8, 128), jnp.float32),
        ],
    )(x_proj, w1p)


def _mlp_compat(ht, a128, bb128, w2p, b2128):
    nblk = V // TCBLKB
    return pl.pallas_call(
        _tc_compat_body,
        grid=(nblk,),
        in_specs=[
            pl.BlockSpec((8, TCBLKB), lambda i: (0, i)),
            pl.BlockSpec((8, 128), lambda i: (0, 0)),
            pl.BlockSpec((8, 128), lambda i: (0, 0)),
            pl.BlockSpec((8, 8), lambda i: (0, 0)),
            pl.BlockSpec((8, 128), lambda i: (0, 0)),
        ],
        out_specs=pl.BlockSpec((8, TCBLKB), lambda i: (0, i)),
        out_shape=jax.ShapeDtypeStruct((8, V), jnp.float32),
    )(ht, a128, bb128, w2p, b2128)


def _rsqrt_scalar(x):
    i = lax.bitcast_convert_type(x, jnp.int32)
    i = jnp.int32(0x5F3759DF) - lax.shift_right_logical(i, 1)
    y = lax.bitcast_convert_type(i, jnp.float32)
    for _ in range(3):
        y = y * (1.5 - 0.5 * x * y * y)
    return y


_SPLAT_DNUMS = lax.GatherDimensionNumbers(
    offset_dims=(), collapsed_slice_dims=(0,), start_index_map=(0,))


def _splat(vec, lane):
    idx = jnp.full((16, 1), lane, jnp.int32)
    return lax.gather(vec, idx, dimension_numbers=_SPLAT_DNUMS,
                      slice_sizes=(1,),
                      mode=lax.GatherScatterMode.PROMISE_IN_BOUNDS)


def _sget(ref, idx):
    return ref[pl.ds(idx, 16)][0]


def _fill_ebuf(ebuf, cbuf, cr, lanes):
    def g_loop(g, _):
        base = g * 16
        idx = (base + lanes) * 8
        for v in range(8):
            plsc.store_scatter(ebuf, (idx + v,), cbuf[v, pl.ds(base, 16)])
        return 0

    lax.fori_loop(0, cr // 16, g_loop, 0)


def _sc_body(csr_hbm, compat_hbm, xmod_hbm, gpar_hbm, out_hbm,
             csr_v, mbuf, outbuf, cbuf1, cbuf2, ebuf, xbuf, gparv):
    wid = lax.axis_index("c") * 16 + lax.axis_index("s")
    _sc_worker(wid, csr_hbm, compat_hbm, xmod_hbm, gpar_hbm, out_hbm,
               csr_v, mbuf, outbuf, cbuf1, cbuf2, ebuf, xbuf, gparv)


def _sc_worker(wid, csr_hbm, compat_hbm, xmod_hbm, gpar_hbm, out_hbm,
               csr_v, mbuf, outbuf, cbuf1, cbuf2, ebuf, xbuf, gparv):
    base_seg = pl.multiple_of(wid * SPW, 64)

    pltpu.sync_copy(csr_hbm.at[pl.ds(base_seg, CSR_COPY)],
                    csr_v.at[pl.ds(0, CSR_COPY)])
    pltpu.sync_copy(gpar_hbm, gparv)

    r0 = _sget(csr_v, 0)
    r1 = _sget(csr_v, SPW)
    a0 = (r0 // 128) * 128
    lanes = lax.iota(jnp.int32, 16)
    mask8 = lanes < 8
    neg = jnp.float32(-jnp.inf)
    minf = jnp.full((16,), neg, jnp.float32)
    zero16 = jnp.zeros((16,), jnp.float32)
    zacc = (zero16,) * 8

    def binsearch(r):
        def step(_, lohi):
            lo, hi = lohi
            mid = lax.shift_right_logical(lo + hi, 1)
            go = _sget(csr_v, mid) <= r
            return (jnp.where(go, mid, lo), jnp.where(go, hi, mid))

        lo, _ = lax.fori_loop(0, 9, step, (jnp.int32(0), jnp.int32(SPW)))
        return lo

    nch1 = (r1 - a0 + (CR1 - 1)) // CR1

    def chunk1(k, carry):
        cursor, m_carry = carry
        off = a0 + k * CR1
        offc = pl.multiple_of(jnp.minimum(off, V - CR1), 128)
        sh = off - offc
        pltpu.sync_copy(compat_hbm.at[:, pl.ds(offc, CR1)], cbuf1)
        _fill_ebuf(ebuf, cbuf1, CR1, lanes)
        nr = jnp.minimum(CR1, r1 - off)
        end = off + nr
        seg_hi = binsearch(end - 1)

        def piece(s2, m_cur):
            a = _sget(csr_v, s2)
            b = _sget(csr_v, s2 + 1)
            a2 = jnp.maximum(a, off)
            b2 = jnp.minimum(b, end)
            m_cur = jnp.where(a >= off, minf, m_cur)

            def rowloop(j, mc):
                cvec = ebuf[pl.ds((j + sh) * 8, 16)]
                return jnp.maximum(mc, jnp.where(mask8, cvec, neg))

            m_cur = lax.fori_loop(a2 - off, b2 - off, rowloop, m_cur)

            @pl.when(b <= end)
            def _():
                mbuf[pl.ds(s2 * 16, 16)] = m_cur

            return m_cur

        m_carry = lax.fori_loop(cursor, seg_hi + 1, piece, m_carry)
        cursor = jnp.where(_sget(csr_v, seg_hi + 1) <= end, seg_hi + 1, seg_hi)
        return (cursor, m_carry)

    _ = lax.fori_loop(0, nch1, chunk1, (jnp.int32(0), minf))

    gw = gparv[pl.ds(0, 16)]
    gb = gparv[pl.ds(16, 16)]

    def finalize(sg, denom, accs, m_c):
        n = _sget(csr_v, sg + 1) - _sget(csr_v, sg)
        m0 = jnp.where(mask8, m_c, 0.0)
        z = jnp.maximum(m0 * gw + gb, 0.0)
        ez = jnp.exp(z + z)
        gate = 1.0 - 2.0 / (ez + 1.0)
        scale = gate / (denom + 1e-12)
        scale = jnp.where(n > 0, scale, 0.0)
        for v in range(8):
            outbuf[sg, pl.ds(v * 16, 16)] = accs[v] * _splat(scale, v)

    nch2 = (r1 - a0 + (CR2 - 1)) // CR2

    def chunk2(k, carry):
        cursor, denom_carry, accs_carry = carry
        off = a0 + k * CR2
        offc = pl.multiple_of(jnp.minimum(off, V - CR2), 128)
        sh = off - offc
        pltpu.sync_copy(compat_hbm.at[:, pl.ds(offc, CR2)], cbuf2)
        pltpu.sync_copy(xmod_hbm.at[pl.ds(offc, CR2), :], xbuf)
        _fill_ebuf(ebuf, cbuf2, CR2, lanes)
        nr = jnp.minimum(CR2, r1 - off)
        end = off + nr
        seg_hi = binsearch(end - 1)

        def piece(s2, c2):
            denom, accs = c2
            a = _sget(csr_v, s2)
            b = _sget(csr_v, s2 + 1)
            a2 = jnp.maximum(a, off)
            b2 = jnp.minimum(b, end)
            fresh = a >= off
            denom = jnp.where(fresh, 0.0, denom)
            accs = tuple(jnp.where(fresh, 0.0, ac) for ac in accs)
            m_c = mbuf[pl.ds(s2 * 16, 16)]
            isn = _rsqrt_scalar(jnp.maximum(b - a, 1).astype(jnp.float32))

            def rowloop(j, c3):
                dn, ac = c3
                jb = j + sh
                cvec = ebuf[pl.ds(jb * 8, 16)]
                e = jnp.exp((cvec - m_c) * isn)
                dn = dn + jnp.where(mask8, e, 0.0)
                ac = tuple(
                    ac[v] + _splat(e, v) * xbuf[jb, pl.ds(v * 16, 16)]
                    for v in range(8))
                return (dn, ac)

            denom, accs = lax.fori_loop(a2 - off, b2 - off, rowloop,
                                        (denom, accs))

            @pl.when(b <= end)
            def _():
                finalize(s2, denom, accs, m_c)

            return (denom, accs)

        denom_carry, accs_carry = lax.fori_loop(cursor, seg_hi + 1, piece,
                                                (denom_carry, accs_carry))
        cursor = jnp.where(_sget(csr_v, seg_hi + 1) <= end, seg_hi + 1, seg_hi)
        return (cursor, denom_carry, accs_carry)

    cursorf, _, _ = lax.fori_loop(0, nch2, chunk2,
                                  (jnp.int32(0), zero16, zacc))

    def tailseg(s2, _):
        finalize(s2, zero16, zacc, zero16)
        return 0

    _ = lax.fori_loop(cursorf, SPW, tailseg, 0)

    pltpu.sync_copy(outbuf, out_hbm.at[pl.ds(base_seg, SPW), :])


def _build_sc_pool(interpret=False):
    mesh = plsc.VectorSubcoreMesh(core_axis_name="c", subcore_axis_name="s")
    return pl.kernel(
        _sc_body,
        out_type=jax.ShapeDtypeStruct((NPAD, F_MOD), jnp.float32),
        mesh=mesh,
        compiler_params=pltpu.CompilerParams(needs_layout_passes=False),
        scratch_types=[
            pltpu.VMEM((CSR_BUF,), jnp.int32),
            pltpu.VMEM((SPW * 16,), jnp.float32),
            pltpu.VMEM((SPW, F_MOD), jnp.float32),
            pltpu.VMEM((8, CR1), jnp.float32),
            pltpu.VMEM((8, CR2), jnp.float32),
            pltpu.VMEM((CR1 * 8 + 16,), jnp.float32),
            pltpu.VMEM((CR2, F_MOD), jnp.float32),
            pltpu.VMEM((32,), jnp.float32),
        ],
        interpret=interpret,
    )


_sc_pool = _build_sc_pool()


def kernel(x_main, x_mod, x_proj, csr_idx, W1, b1, bn_gamma, bn_beta,
           W2, b2, gate_w, gate_b):
    del x_main
    del b1
    w1p = jnp.zeros((8, F_PROJ), jnp.float32).at[:4].set(W1)
    ht, stats = _mlp_h_stats(x_proj, w1p)
    mu = stats[:, 0] / V
    var = stats[:, 1] / V - mu * mu
    gam = jnp.zeros((8,), jnp.float32).at[:4].set(bn_gamma)
    bet = jnp.zeros((8,), jnp.float32).at[:4].set(bn_beta)
    a = gam * lax.rsqrt(var + EPS_BN)
    bb = bet - mu * a
    w2p = jnp.zeros((8, 8), jnp.float32).at[:, :4].set(W2)
    tile128 = lambda v: jnp.tile(v.reshape(8, 1), (1, 128))
    b2p = jnp.zeros((8,), jnp.float32).at[:G].set(b2)
    compatT = _mlp_compat(ht, tile128(a), tile128(bb), w2p, tile128(b2p))

    csr32 = csr_idx.astype(jnp.int32)
    csr_pad = jnp.concatenate(
        [csr32, jnp.full(((NW - 1) * SPW + CSR_COPY - (N + 1),), V, jnp.int32)])
    gpar = jnp.concatenate([
        jnp.tile(gate_w.reshape(-1), 2).astype(jnp.float32),
        jnp.tile(gate_b.reshape(-1), 2).astype(jnp.float32),
    ])
    out = _sc_pool(csr_pad, compatT, x_mod, gpar)
    x_pool = out[:N]
    x_seen = csr_idx[1:] > csr_idx[:-1]
    return x_pool, x_seen


# speedup vs baseline: 14.3214x; 1.1794x over previous
import jax
import jax.numpy as jnp
from jax import lax
from jax.experimental import pallas as pl
from jax.experimental.pallas import tpu as pltpu
from jax.experimental.pallas import tpu_sc as plsc

N = 10000
V = 320000
F_PROJ = 64
F_MOD = 128
G = 8
EPS_BN = 1e-5

NW = 32
SPW = 320
NPAD = NW * SPW
CSR_COPY = 328
CSR_BUF = 344
CR1 = 256
CR2 = 256
TCBLK = 2560
TCBLKB = 12800


def _tc_h_stats_body(xp_ref, w1_ref, ht_ref, stats_ref):
    i = pl.program_id(0)
    x = xp_ref[...]
    ht = lax.dot_general(w1_ref[...], x, (((1,), (1,)), ((), ())),
                         preferred_element_type=jnp.float32)
    ht_ref[...] = ht
    s1 = jnp.sum(ht, axis=1, keepdims=True)
    s2 = jnp.sum(ht * ht, axis=1, keepdims=True)
    upd = jnp.pad(jnp.concatenate([s1, s2], axis=1), ((0, 0), (0, 126)))

    @pl.when(i == 0)
    def _():
        stats_ref[...] = upd

    @pl.when(i > 0)
    def _():
        stats_ref[...] += upd


def _tc_compat_body(ht_ref, a_ref, bb_ref, w2_ref, b2_ref, out_ref):
    a = a_ref[...][:, 0:1]
    bb = bb_ref[...][:, 0:1]
    b2 = b2_ref[...][:, 0:1]
    hn = jnp.maximum(ht_ref[...] * a + bb, 0.0)
    out_ref[...] = lax.dot_general(w2_ref[...], hn, (((1,), (0,)), ((), ())),
                                   preferred_element_type=jnp.float32) + b2


def _mlp_h_stats(x_proj, w1p):
    nblk = V // TCBLK
    return pl.pallas_call(
        _tc_h_stats_body,
        grid=(nblk,),
        in_specs=[
            pl.BlockSpec((TCBLK, F_PROJ), lambda i: (i, 0)),
            pl.BlockSpec((8, F_PROJ), lambda i: (0, 0)),
        ],
        out_specs=[
            pl.BlockSpec((8, TCBLK), lambda i: (0, i)),
            pl.BlockSpec((8, 128), lambda i: (0, 0)),
        ],
        out_shape=[
            jax.ShapeDtypeStruct((8, V), jnp.float32),
            jax.ShapeDtypeStruct((8, 128), jnp.float32),
        ],
    )(x_proj, w1p)


def _mlp_compat(ht, a128, bb128, w2p, b2128):
    nblk = V // TCBLKB
    return pl.pallas_call(
        _tc_compat_body,
        grid=(nblk,),
        in_specs=[
            pl.BlockSpec((8, TCBLKB), lambda i: (0, i)),
            pl.BlockSpec((8, 128), lambda i: (0, 0)),
            pl.BlockSpec((8, 128), lambda i: (0, 0)),
            pl.BlockSpec((8, 8), lambda i: (0, 0)),
            pl.BlockSpec((8, 128), lambda i: (0, 0)),
        ],
        out_specs=pl.BlockSpec((8, TCBLKB), lambda i: (0, i)),
        out_shape=jax.ShapeDtypeStruct((8, V), jnp.float32),
    )(ht, a128, bb128, w2p, b2128)


def _rsqrt_scalar(x):
    i = lax.bitcast_convert_type(x, jnp.int32)
    i = jnp.int32(0x5F3759DF) - lax.shift_right_logical(i, 1)
    y = lax.bitcast_convert_type(i, jnp.float32)
    for _ in range(3):
        y = y * (1.5 - 0.5 * x * y * y)
    return y


_SPLAT_DNUMS = lax.GatherDimensionNumbers(
    offset_dims=(), collapsed_slice_dims=(0,), start_index_map=(0,))


def _splat(vec, lane):
    idx = jnp.full((16, 1), lane, jnp.int32)
    return lax.gather(vec, idx, dimension_numbers=_SPLAT_DNUMS,
                      slice_sizes=(1,),
                      mode=lax.GatherScatterMode.PROMISE_IN_BOUNDS)


def _sget(ref, idx):
    return ref[pl.ds(idx, 16)][0]


def _fill_ebuf(ebuf, cbuf, cr, lanes):
    def g_loop(g, _):
        base = g * 16
        idx = (base + lanes) * 8
        for v in range(8):
            plsc.store_scatter(ebuf, (idx + v,), cbuf[v, pl.ds(base, 16)])
        return 0

    lax.fori_loop(0, cr // 16, g_loop, 0)


def _sc_body(csr_hbm, compat_hbm, xmod_hbm, gpar_hbm, out_hbm,
             csr_v, mbuf, outbuf, cbuf1, cbufA, cbufB, ebuf, xbufA, xbufB,
             gparv, semA, semB):
    wid = lax.axis_index("c") * 16 + lax.axis_index("s")
    _sc_worker(wid, csr_hbm, compat_hbm, xmod_hbm, gpar_hbm, out_hbm,
               csr_v, mbuf, outbuf, cbuf1, cbufA, cbufB, ebuf, xbufA, xbufB,
               gparv, semA, semB)


def _sc_worker(wid, csr_hbm, compat_hbm, xmod_hbm, gpar_hbm, out_hbm,
               csr_v, mbuf, outbuf, cbuf1, cbufA, cbufB, ebuf, xbufA, xbufB,
               gparv, semA, semB):
    base_seg = pl.multiple_of(wid * SPW, 64)

    pltpu.sync_copy(csr_hbm.at[pl.ds(base_seg, CSR_COPY)],
                    csr_v.at[pl.ds(0, CSR_COPY)])
    pltpu.sync_copy(gpar_hbm, gparv)

    r0 = _sget(csr_v, 0)
    r1 = _sget(csr_v, SPW)
    a0 = (r0 // 128) * 128
    lanes = lax.iota(jnp.int32, 16)
    mask8 = lanes < 8
    neg = jnp.float32(-jnp.inf)
    minf = jnp.full((16,), neg, jnp.float32)
    zero16 = jnp.zeros((16,), jnp.float32)
    zacc = (zero16,) * 8

    def binsearch(r):
        def step(_, lohi):
            lo, hi = lohi
            mid = lax.shift_right_logical(lo + hi, 1)
            go = _sget(csr_v, mid) <= r
            return (jnp.where(go, mid, lo), jnp.where(go, hi, mid))

        lo, _ = lax.fori_loop(0, 9, step, (jnp.int32(0), jnp.int32(SPW)))
        return lo

    nch1 = (r1 - a0 + (CR1 - 1)) // CR1

    def chunk1(k, carry):
        cursor, m_carry = carry
        off = a0 + k * CR1
        offc = pl.multiple_of(jnp.minimum(off, V - CR1), 128)
        sh = off - offc
        pltpu.sync_copy(compat_hbm.at[:, pl.ds(offc, CR1)], cbuf1)
        _fill_ebuf(ebuf, cbuf1, CR1, lanes)
        nr = jnp.minimum(CR1, r1 - off)
        end = off + nr
        seg_hi = binsearch(end - 1)

        def piece(s2, m_cur):
            a = _sget(csr_v, s2)
            b = _sget(csr_v, s2 + 1)
            a2 = jnp.maximum(a, off)
            b2 = jnp.minimum(b, end)
            m_cur = jnp.where(a >= off, minf, m_cur)

            def rowloop(j, mc):
                cvec = ebuf[pl.ds((j + sh) * 8, 16)]
                return jnp.maximum(mc, jnp.where(mask8, cvec, neg))

            m_cur = lax.fori_loop(a2 - off, b2 - off, rowloop, m_cur)

            @pl.when(b <= end)
            def _():
                mbuf[pl.ds(s2 * 16, 16)] = m_cur

            return m_cur

        m_carry = lax.fori_loop(cursor, seg_hi + 1, piece, m_carry)
        cursor = jnp.where(_sget(csr_v, seg_hi + 1) <= end, seg_hi + 1, seg_hi)
        return (cursor, m_carry)

    _ = lax.fori_loop(0, nch1, chunk1, (jnp.int32(0), minf))

    gw = gparv[pl.ds(0, 16)]
    gb = gparv[pl.ds(16, 16)]

    def finalize(sg, denom, accs, m_c):
        n = _sget(csr_v, sg + 1) - _sget(csr_v, sg)
        m0 = jnp.where(mask8, m_c, 0.0)
        z = jnp.maximum(m0 * gw + gb, 0.0)
        ez = jnp.exp(z + z)
        gate = 1.0 - 2.0 / (ez + 1.0)
        scale = gate / (denom + 1e-12)
        scale = jnp.where(n > 0, scale, 0.0)
        for v in range(8):
            outbuf[sg, pl.ds(v * 16, 16)] = accs[v] * _splat(scale, v)

    nch2 = (r1 - a0 + (CR2 - 1)) // CR2
    bufs = ((cbufA, xbufA, semA), (cbufB, xbufB, semB))

    def chunk_off(c):
        off = a0 + c * CR2
        offc = pl.multiple_of(jnp.minimum(off, V - CR2), 128)
        return off, offc

    def start_dma(c, cbuf, xbuf, sem):
        _, offc = chunk_off(c)
        pltpu.async_copy(compat_hbm.at[:, pl.ds(offc, CR2)], cbuf, sem)
        pltpu.async_copy(xmod_hbm.at[pl.ds(offc, CR2), :], xbuf, sem)

    def wait_dma(c, cbuf, xbuf, sem):
        _, offc = chunk_off(c)
        pltpu.make_async_copy(compat_hbm.at[:, pl.ds(offc, CR2)], cbuf,
                              sem).wait()
        pltpu.make_async_copy(xmod_hbm.at[pl.ds(offc, CR2), :], xbuf,
                              sem).wait()

    @pl.when(nch2 > 0)
    def _():
        start_dma(jnp.int32(0), *bufs[0])

    def chunk2(k, carry, p, cbuf2, xbuf, sem):
        cursor, denom_carry, accs_carry = carry
        guard = k < nch2
        off, offc = chunk_off(k)
        sh = off - offc

        @pl.when(guard)
        def _():
            wait_dma(k, cbuf2, xbuf, sem)

        @pl.when(k + 1 < nch2)
        def _():
            start_dma(k + 1, *bufs[1 - p])

        @pl.when(guard)
        def _():
            _fill_ebuf(ebuf, cbuf2, CR2, lanes)

        nr = jnp.where(guard, jnp.minimum(CR2, r1 - off), 0)
        end = off + nr
        seg_hi = jnp.where(guard, binsearch(end - 1), cursor - 1)

        def piece(s2, c2):
            denom, accs = c2
            a = _sget(csr_v, s2)
            b = _sget(csr_v, s2 + 1)
            a2 = jnp.maximum(a, off)
            b2 = jnp.minimum(b, end)
            fresh = a >= off
            denom = jnp.where(fresh, 0.0, denom)
            accs = tuple(jnp.where(fresh, 0.0, ac) for ac in accs)
            m_c = mbuf[pl.ds(s2 * 16, 16)]
            isn = _rsqrt_scalar(jnp.maximum(b - a, 1).astype(jnp.float32))

            def rowloop(j, c3):
                dn, ac = c3
                jb = j + sh
                cvec = ebuf[pl.ds(jb * 8, 16)]
                e = jnp.exp((cvec - m_c) * isn)
                dn = dn + jnp.where(mask8, e, 0.0)
                ac = tuple(
                    ac[v] + _splat(e, v) * xbuf[jb, pl.ds(v * 16, 16)]
                    for v in range(8))
                return (dn, ac)

            denom, accs = lax.fori_loop(a2 - off, b2 - off, rowloop,
                                        (denom, accs))

            @pl.when(b <= end)
            def _():
                finalize(s2, denom, accs, m_c)

            return (denom, accs)

        denom_carry, accs_carry = lax.fori_loop(cursor, seg_hi + 1, piece,
                                                (denom_carry, accs_carry))
        cursor = jnp.where(_sget(csr_v, seg_hi + 1) <= end, seg_hi + 1, seg_hi)
        return (cursor, denom_carry, accs_carry)

    def chunk_pair(k2, carry):
        for p in range(2):
            cbuf_p, xbuf_p, sem_p = bufs[p]
            carry = chunk2(k2 * 2 + p, carry, p, cbuf_p, xbuf_p, sem_p)
        return carry

    cursorf, _, _ = lax.fori_loop(0, (nch2 + 1) // 2, chunk_pair,
                                  (jnp.int32(0), zero16, zacc))

    def tailseg(s2, _):
        finalize(s2, zero16, zacc, zero16)
        return 0

    _ = lax.fori_loop(cursorf, SPW, tailseg, 0)

    pltpu.sync_copy(outbuf, out_hbm.at[pl.ds(base_seg, SPW), :])


def _build_sc_pool(interpret=False):
    mesh = plsc.VectorSubcoreMesh(core_axis_name="c", subcore_axis_name="s")
    return pl.kernel(
        _sc_body,
        out_type=jax.ShapeDtypeStruct((NPAD, F_MOD), jnp.float32),
        mesh=mesh,
        compiler_params=pltpu.CompilerParams(needs_layout_passes=False),
        scratch_types=[
            pltpu.VMEM((CSR_BUF,), jnp.int32),
            pltpu.VMEM((SPW * 16,), jnp.float32),
            pltpu.VMEM((SPW, F_MOD), jnp.float32),
            pltpu.VMEM((8, CR1), jnp.float32),
            pltpu.VMEM((8, CR2), jnp.float32),
            pltpu.VMEM((8, CR2), jnp.float32),
            pltpu.VMEM((max(CR1, CR2) * 8 + 16,), jnp.float32),
            pltpu.VMEM((CR2, F_MOD), jnp.float32),
            pltpu.VMEM((CR2, F_MOD), jnp.float32),
            pltpu.VMEM((32,), jnp.float32),
            pltpu.SemaphoreType.DMA,
            pltpu.SemaphoreType.DMA,
        ],
        interpret=interpret,
    )


_sc_pool = _build_sc_pool()


def kernel(x_main, x_mod, x_proj, csr_idx, W1, b1, bn_gamma, bn_beta,
           W2, b2, gate_w, gate_b):
    del x_main
    del b1
    w1p = jnp.zeros((8, F_PROJ), jnp.float32).at[:4].set(W1)
    ht, stats = _mlp_h_stats(x_proj, w1p)
    mu = stats[:, 0] / V
    var = stats[:, 1] / V - mu * mu
    gam = jnp.zeros((8,), jnp.float32).at[:4].set(bn_gamma)
    bet = jnp.zeros((8,), jnp.float32).at[:4].set(bn_beta)
    a = gam * lax.rsqrt(var + EPS_BN)
    bb = bet - mu * a
    w2p = jnp.zeros((8, 8), jnp.float32).at[:, :4].set(W2)
    tile128 = lambda v: jnp.tile(v.reshape(8, 1), (1, 128))
    b2p = jnp.zeros((8,), jnp.float32).at[:G].set(b2)
    compatT = _mlp_compat(ht, tile128(a), tile128(bb), w2p, tile128(b2p))

    csr32 = csr_idx.astype(jnp.int32)
    csr_pad = jnp.concatenate(
        [csr32, jnp.full(((NW - 1) * SPW + CSR_COPY - (N + 1),), V, jnp.int32)])
    gpar = jnp.concatenate([
        jnp.tile(gate_w.reshape(-1), 2).astype(jnp.float32),
        jnp.tile(gate_b.reshape(-1), 2).astype(jnp.float32),
    ])
    out = _sc_pool(csr_pad, compatT, x_mod, gpar)
    x_pool = out[:N]
    x_seen = csr_idx[1:] > csr_idx[:-1]
    return x_pool, x_seen


# speedup vs baseline: 14.3237x; 1.0002x over previous
import jax
import jax.numpy as jnp
from jax import lax
from jax.experimental import pallas as pl
from jax.experimental.pallas import tpu as pltpu
from jax.experimental.pallas import tpu_sc as plsc

N = 10000
V = 320000
F_PROJ = 64
F_MOD = 128
G = 8
EPS_BN = 1e-5

NW = 32
SPW = 320
NPAD = NW * SPW
CSR_COPY = 328
CSR_BUF = 344
CR1 = 256
CR2 = 256
TCBLK = 2560
TCBLKB = 12800


def _tc_h_stats_body(xp_ref, w1_ref, ht_ref, stats_ref):
    i = pl.program_id(0)
    x = xp_ref[...]
    ht = lax.dot_general(w1_ref[...], x, (((1,), (1,)), ((), ())),
                         preferred_element_type=jnp.float32)
    ht_ref[...] = ht
    s1 = jnp.sum(ht, axis=1, keepdims=True)
    s2 = jnp.sum(ht * ht, axis=1, keepdims=True)
    upd = jnp.pad(jnp.concatenate([s1, s2], axis=1), ((0, 0), (0, 126)))

    @pl.when(i == 0)
    def _():
        stats_ref[...] = upd

    @pl.when(i > 0)
    def _():
        stats_ref[...] += upd


def _tc_compat_body(ht_ref, a_ref, bb_ref, w2_ref, b2_ref, out_ref):
    a = a_ref[...][:, 0:1]
    bb = bb_ref[...][:, 0:1]
    b2 = b2_ref[...][:, 0:1]
    hn = jnp.maximum(ht_ref[...] * a + bb, 0.0)
    out_ref[...] = lax.dot_general(w2_ref[...], hn, (((1,), (0,)), ((), ())),
                                   preferred_element_type=jnp.float32) + b2


def _mlp_h_stats(x_proj, w1p):
    nblk = V // TCBLK
    return pl.pallas_call(
        _tc_h_stats_body,
        grid=(nblk,),
        in_specs=[
            pl.BlockSpec((TCBLK, F_PROJ), lambda i: (i, 0)),
            pl.BlockSpec((8, F_PROJ), lambda i: (0, 0)),
        ],
        out_specs=[
            pl.BlockSpec((8, TCBLK), lambda i: (0, i)),
            pl.BlockSpec((8, 128), lambda i: (0, 0)),
        ],
        out_shape=[
            jax.ShapeDtypeStruct((8, V), jnp.float32),
            jax.ShapeDtypeStruct((8, 128), jnp.float32),
        ],
    )(x_proj, w1p)


def _mlp_compat(ht, a128, bb128, w2p, b2128):
    nblk = V // TCBLKB
    return pl.pallas_call(
        _tc_compat_body,
        grid=(nblk,),
        in_specs=[
            pl.BlockSpec((8, TCBLKB), lambda i: (0, i)),
            pl.BlockSpec((8, 128), lambda i: (0, 0)),
            pl.BlockSpec((8, 128), lambda i: (0, 0)),
            pl.BlockSpec((8, 8), lambda i: (0, 0)),
            pl.BlockSpec((8, 128), lambda i: (0, 0)),
        ],
        out_specs=pl.BlockSpec((8, TCBLKB), lambda i: (0, i)),
        out_shape=jax.ShapeDtypeStruct((8, V), jnp.float32),
    )(ht, a128, bb128, w2p, b2128)


def _rsqrt_scalar(x):
    i = lax.bitcast_convert_type(x, jnp.int32)
    i = jnp.int32(0x5F3759DF) - lax.shift_right_logical(i, 1)
    y = lax.bitcast_convert_type(i, jnp.float32)
    for _ in range(3):
        y = y * (1.5 - 0.5 * x * y * y)
    return y


_SPLAT_DNUMS = lax.GatherDimensionNumbers(
    offset_dims=(), collapsed_slice_dims=(0,), start_index_map=(0,))


def _splat(vec, lane):
    idx = jnp.full((16, 1), lane, jnp.int32)
    return lax.gather(vec, idx, dimension_numbers=_SPLAT_DNUMS,
                      slice_sizes=(1,),
                      mode=lax.GatherScatterMode.PROMISE_IN_BOUNDS)


def _sget(ref, idx):
    return ref[pl.ds(idx, 16)][0]


def _fill_ebuf(ebuf, cbuf, cr, lanes):
    def g_loop(g, _):
        base = g * 16
        idx = (base + lanes) * 8
        for v in range(8):
            plsc.store_scatter(ebuf, (idx + v,), cbuf[v, pl.ds(base, 16)])
        return 0

    lax.fori_loop(0, cr // 16, g_loop, 0)


def _sc_body(csr_hbm, compat_hbm, xmod_hbm, gpar_hbm, out_hbm,
             csr_v, mbuf, outbuf, cbuf1, cbufA, cbufB, ebuf, xbufA, xbufB,
             gparv, semA, semB):
    wid = lax.axis_index("c") * 16 + lax.axis_index("s")
    _sc_worker(wid, csr_hbm, compat_hbm, xmod_hbm, gpar_hbm, out_hbm,
               csr_v, mbuf, outbuf, cbuf1, cbufA, cbufB, ebuf, xbufA, xbufB,
               gparv, semA, semB)


def _sc_worker(wid, csr_hbm, compat_hbm, xmod_hbm, gpar_hbm, out_hbm,
               csr_v, mbuf, outbuf, cbuf1, cbufA, cbufB, ebuf, xbufA, xbufB,
               gparv, semA, semB):
    base_seg = pl.multiple_of(wid * SPW, 64)

    pltpu.sync_copy(csr_hbm.at[pl.ds(base_seg, CSR_COPY)],
                    csr_v.at[pl.ds(0, CSR_COPY)])
    pltpu.sync_copy(gpar_hbm, gparv)

    r0 = _sget(csr_v, 0)
    r1 = _sget(csr_v, SPW)
    a0 = (r0 // 128) * 128
    lanes = lax.iota(jnp.int32, 16)
    mask8 = lanes < 8
    neg = jnp.float32(-jnp.inf)
    minf = jnp.full((16,), neg, jnp.float32)
    zero16 = jnp.zeros((16,), jnp.float32)
    zacc = (zero16,) * 8

    def binsearch(r):
        def step(_, lohi):
            lo, hi = lohi
            mid = lax.shift_right_logical(lo + hi, 1)
            go = _sget(csr_v, mid) <= r
            return (jnp.where(go, mid, lo), jnp.where(go, hi, mid))

        lo, _ = lax.fori_loop(0, 9, step, (jnp.int32(0), jnp.int32(SPW)))
        return lo

    nch1 = (r1 - a0 + (CR1 - 1)) // CR1

    def chunk1(k, carry):
        cursor, m_carry = carry
        off = a0 + k * CR1
        offc = pl.multiple_of(jnp.minimum(off, V - CR1), 128)
        sh = off - offc
        pltpu.sync_copy(compat_hbm.at[:, pl.ds(offc, CR1)], cbuf1)
        _fill_ebuf(ebuf, cbuf1, CR1, lanes)
        nr = jnp.minimum(CR1, r1 - off)
        end = off + nr
        seg_hi = binsearch(end - 1)

        def piece(s2, m_cur):
            a = _sget(csr_v, s2)
            b = _sget(csr_v, s2 + 1)
            a2 = jnp.maximum(a, off)
            b2 = jnp.minimum(b, end)
            m_cur = jnp.where(a >= off, minf, m_cur)

            def rowloop(j, mc):
                cvec = ebuf[pl.ds((j + sh) * 8, 16)]
                return jnp.maximum(mc, jnp.where(mask8, cvec, neg))

            m_cur = lax.fori_loop(a2 - off, b2 - off, rowloop, m_cur)

            @pl.when(b <= end)
            def _():
                mbuf[pl.ds(s2 * 16, 16)] = m_cur

            return m_cur

        m_carry = lax.fori_loop(cursor, seg_hi + 1, piece, m_carry)
        cursor = jnp.where(_sget(csr_v, seg_hi + 1) <= end, seg_hi + 1, seg_hi)
        return (cursor, m_carry)

    _ = lax.fori_loop(0, nch1, chunk1, (jnp.int32(0), minf))

    gw = gparv[pl.ds(0, 16)]
    gb = gparv[pl.ds(16, 16)]

    def finalize(sg, denom, accs, m_c):
        n = _sget(csr_v, sg + 1) - _sget(csr_v, sg)
        m0 = jnp.where(mask8, m_c, 0.0)
        z = jnp.maximum(m0 * gw + gb, 0.0)
        ez = jnp.exp(z + z)
        gate = 1.0 - 2.0 / (ez + 1.0)
        scale = gate / (denom + 1e-12)
        scale = jnp.where(n > 0, scale, 0.0)
        for v in range(8):
            outbuf[sg, pl.ds(v * 16, 16)] = accs[v] * _splat(scale, v)

    nch2 = (r1 - a0 + (CR2 - 1)) // CR2
    bufs = ((cbufA, xbufA, semA), (cbufB, xbufB, semB))

    def chunk_off(c):
        off = a0 + c * CR2
        offc = pl.multiple_of(jnp.minimum(off, V - CR2), 128)
        return off, offc

    def start_dma(c, cbuf, xbuf, sem):
        _, offc = chunk_off(c)
        pltpu.async_copy(compat_hbm.at[:, pl.ds(offc, CR2)], cbuf, sem)
        pltpu.async_copy(xmod_hbm.at[pl.ds(offc, CR2), :], xbuf, sem)

    def wait_dma(c, cbuf, xbuf, sem):
        _, offc = chunk_off(c)
        pltpu.make_async_copy(compat_hbm.at[:, pl.ds(offc, CR2)], cbuf,
                              sem).wait()
        pltpu.make_async_copy(xmod_hbm.at[pl.ds(offc, CR2), :], xbuf,
                              sem).wait()

    @pl.when(nch2 > 0)
    def _():
        start_dma(jnp.int32(0), *bufs[0])

    def chunk2(k, carry, p, cbuf2, xbuf, sem):
        cursor, denom_carry, accs_carry = carry
        guard = k < nch2
        off, offc = chunk_off(k)
        sh = off - offc

        @pl.when(guard)
        def _():
            wait_dma(k, cbuf2, xbuf, sem)

        @pl.when(k + 1 < nch2)
        def _():
            start_dma(k + 1, *bufs[1 - p])

        @pl.when(guard)
        def _():
            _fill_ebuf(ebuf, cbuf2, CR2, lanes)

        nr = jnp.where(guard, jnp.minimum(CR2, r1 - off), 0)
        end = off + nr
        seg_hi = jnp.where(guard, binsearch(end - 1), cursor - 1)

        def piece(s2, c2):
            denom, accs = c2
            a = _sget(csr_v, s2)
            b = _sget(csr_v, s2 + 1)
            a2 = jnp.maximum(a, off)
            b2 = jnp.minimum(b, end)
            fresh = a >= off
            denom = jnp.where(fresh, 0.0, denom)
            accs = tuple(jnp.where(fresh, 0.0, ac) for ac in accs)
            m_c = mbuf[pl.ds(s2 * 16, 16)]
            isn = _rsqrt_scalar(jnp.maximum(b - a, 1).astype(jnp.float32))

            def rowloop(j, c3):
                dn, ac = c3
                jb = j + sh
                cvec = ebuf[pl.ds(jb * 8, 16)]
                e = jnp.exp((cvec - m_c) * isn)
                dn = dn + jnp.where(mask8, e, 0.0)
                ac = tuple(
                    ac[v] + _splat(e, v) * xbuf[jb, pl.ds(v * 16, 16)]
                    for v in range(8))
                return (dn, ac)

            denom, accs = lax.fori_loop(a2 - off, b2 - off, rowloop,
                                        (denom, accs))

            @pl.when(b <= end)
            def _():
                finalize(s2, denom, accs, m_c)

            return (denom, accs)

        denom_carry, accs_carry = lax.fori_loop(cursor, seg_hi + 1, piece,
                                                (denom_carry, accs_carry))
        cursor = jnp.where(_sget(csr_v, seg_hi + 1) <= end, seg_hi + 1, seg_hi)
        return (cursor, denom_carry, accs_carry)

    def chunk_pair(k2, carry):
        for p in range(2):
            cbuf_p, xbuf_p, sem_p = bufs[p]
            carry = chunk2(k2 * 2 + p, carry, p, cbuf_p, xbuf_p, sem_p)
        return carry

    cursorf, _, _ = lax.fori_loop(0, (nch2 + 1) // 2, chunk_pair,
                                  (jnp.int32(0), zero16, zacc))

    def tailseg(s2, _):
        finalize(s2, zero16, zacc, zero16)
        return 0

    _ = lax.fori_loop(cursorf, SPW, tailseg, 0)

    pltpu.sync_copy(outbuf, out_hbm.at[pl.ds(base_seg, SPW), :])


def _build_sc_pool(interpret=False):
    mesh = plsc.VectorSubcoreMesh(core_axis_name="c", subcore_axis_name="s")
    return pl.kernel(
        _sc_body,
        out_type=jax.ShapeDtypeStruct((NPAD, F_MOD), jnp.float32),
        mesh=mesh,
        compiler_params=pltpu.CompilerParams(needs_layout_passes=False,
                                             use_tc_tiling_on_sc=True),
        scratch_types=[
            pltpu.VMEM((CSR_BUF,), jnp.int32),
            pltpu.VMEM((SPW * 16,), jnp.float32),
            pltpu.VMEM((SPW, F_MOD), jnp.float32),
            pltpu.VMEM((8, CR1), jnp.float32),
            pltpu.VMEM((8, CR2), jnp.float32),
            pltpu.VMEM((8, CR2), jnp.float32),
            pltpu.VMEM((max(CR1, CR2) * 8 + 16,), jnp.float32),
            pltpu.VMEM((CR2, F_MOD), jnp.float32),
            pltpu.VMEM((CR2, F_MOD), jnp.float32),
            pltpu.VMEM((32,), jnp.float32),
            pltpu.SemaphoreType.DMA,
            pltpu.SemaphoreType.DMA,
        ],
        interpret=interpret,
    )


_sc_pool = _build_sc_pool()


def kernel(x_main, x_mod, x_proj, csr_idx, W1, b1, bn_gamma, bn_beta,
           W2, b2, gate_w, gate_b):
    del x_main
    del b1
    w1p = jnp.zeros((8, F_PROJ), jnp.float32).at[:4].set(W1)
    ht, stats = _mlp_h_stats(x_proj, w1p)
    mu = stats[:, 0] / V
    var = stats[:, 1] / V - mu * mu
    gam = jnp.zeros((8,), jnp.float32).at[:4].set(bn_gamma)
    bet = jnp.zeros((8,), jnp.float32).at[:4].set(bn_beta)
    a = gam * lax.rsqrt(var + EPS_BN)
    bb = bet - mu * a
    w2p = jnp.zeros((8, 8), jnp.float32).at[:, :4].set(W2)
    tile128 = lambda v: jnp.tile(v.reshape(8, 1), (1, 128))
    b2p = jnp.zeros((8,), jnp.float32).at[:G].set(b2)
    compatT = _mlp_compat(ht, tile128(a), tile128(bb), w2p, tile128(b2p))

    csr32 = csr_idx.astype(jnp.int32)
    csr_pad = jnp.concatenate(
        [csr32, jnp.full(((NW - 1) * SPW + CSR_COPY - (N + 1),), V, jnp.int32)])
    gpar = jnp.concatenate([
        jnp.tile(gate_w.reshape(-1), 2).astype(jnp.float32),
        jnp.tile(gate_b.reshape(-1), 2).astype(jnp.float32),
    ])
    out = _sc_pool(csr_pad, compatT, x_mod, gpar)
    x_pool = out[:N]
    x_seen = csr_idx[1:] > csr_idx[:-1]
    return x_pool, x_seen


# speedup vs baseline: 19.0724x; 1.3315x over previous
import jax
import jax.numpy as jnp
from jax import lax
from jax.experimental import pallas as pl
from jax.experimental.pallas import tpu as pltpu
from jax.experimental.pallas import tpu_sc as plsc

N = 10000
V = 320000
F_PROJ = 64
F_MOD = 128
G = 8
EPS_BN = 1e-5

NW = 32
SPW = 320
NPAD = NW * SPW
CSR_COPY = 328
CSR_BUF = 344
CR1 = 256
CR2 = 256
TCBLK = 2560
TCBLKB = 12800


def _tc_h_stats_body(xp_ref, w1_ref, ht_ref, stats_ref):
    i = pl.program_id(0)
    x = xp_ref[...]
    ht = lax.dot_general(w1_ref[...], x, (((1,), (0,)), ((), ())),
                         preferred_element_type=jnp.float32)
    ht_ref[...] = ht
    s1 = jnp.sum(ht, axis=1, keepdims=True)
    s2 = jnp.sum(ht * ht, axis=1, keepdims=True)
    upd = jnp.pad(jnp.concatenate([s1, s2], axis=1), ((0, 0), (0, 126)))

    @pl.when(i == 0)
    def _():
        stats_ref[...] = upd

    @pl.when(i > 0)
    def _():
        stats_ref[...] += upd


def _tc_compat_body(ht_ref, a_ref, bb_ref, w2_ref, b2_ref, out_ref):
    a = a_ref[...][:, 0:1]
    bb = bb_ref[...][:, 0:1]
    b2 = b2_ref[...][:, 0:1]
    hn = jnp.maximum(ht_ref[...] * a + bb, 0.0)
    out_ref[...] = lax.dot_general(w2_ref[...], hn, (((1,), (0,)), ((), ())),
                                   preferred_element_type=jnp.float32) + b2


def _mlp_h_stats(xpt, w1p):
    nblk = V // TCBLK
    return pl.pallas_call(
        _tc_h_stats_body,
        grid=(nblk,),
        in_specs=[
            pl.BlockSpec((F_PROJ, TCBLK), lambda i: (0, i)),
            pl.BlockSpec((8, F_PROJ), lambda i: (0, 0)),
        ],
        out_specs=[
            pl.BlockSpec((8, TCBLK), lambda i: (0, i)),
            pl.BlockSpec((8, 128), lambda i: (0, 0)),
        ],
        out_shape=[
            jax.ShapeDtypeStruct((8, V), jnp.float32),
            jax.ShapeDtypeStruct((8, 128), jnp.float32),
        ],
    )(xpt, w1p)


def _mlp_compat(ht, a128, bb128, w2p, b2128):
    nblk = V // TCBLKB
    return pl.pallas_call(
        _tc_compat_body,
        grid=(nblk,),
        in_specs=[
            pl.BlockSpec((8, TCBLKB), lambda i: (0, i)),
            pl.BlockSpec((8, 128), lambda i: (0, 0)),
            pl.BlockSpec((8, 128), lambda i: (0, 0)),
            pl.BlockSpec((8, 8), lambda i: (0, 0)),
            pl.BlockSpec((8, 128), lambda i: (0, 0)),
        ],
        out_specs=pl.BlockSpec((8, TCBLKB), lambda i: (0, i)),
        out_shape=jax.ShapeDtypeStruct((8, V), jnp.float32),
    )(ht, a128, bb128, w2p, b2128)


def _rsqrt_scalar(x):
    i = lax.bitcast_convert_type(x, jnp.int32)
    i = jnp.int32(0x5F3759DF) - lax.shift_right_logical(i, 1)
    y = lax.bitcast_convert_type(i, jnp.float32)
    for _ in range(3):
        y = y * (1.5 - 0.5 * x * y * y)
    return y


_SPLAT_DNUMS = lax.GatherDimensionNumbers(
    offset_dims=(), collapsed_slice_dims=(0,), start_index_map=(0,))


def _splat(vec, lane):
    idx = jnp.full((16, 1), lane, jnp.int32)
    return lax.gather(vec, idx, dimension_numbers=_SPLAT_DNUMS,
                      slice_sizes=(1,),
                      mode=lax.GatherScatterMode.PROMISE_IN_BOUNDS)


def _sget(ref, idx):
    return ref[pl.ds(idx, 16)][0]


def _fill_ebuf(ebuf, cbuf, cr, lanes):
    def g_loop(g, _):
        base = g * 16
        idx = (base + lanes) * 8
        for v in range(8):
            plsc.store_scatter(ebuf, (idx + v,), cbuf[v, pl.ds(base, 16)])
        return 0

    lax.fori_loop(0, cr // 16, g_loop, 0)


def _sc_body(csr_hbm, compat_hbm, xmod_hbm, gpar_hbm, out_hbm,
             csr_v, mbuf, outbuf, cbuf1, cbufA, cbufB, ebuf, xbufA, xbufB,
             gparv, semA, semB):
    wid = lax.axis_index("c") * 16 + lax.axis_index("s")
    _sc_worker(wid, csr_hbm, compat_hbm, xmod_hbm, gpar_hbm, out_hbm,
               csr_v, mbuf, outbuf, cbuf1, cbufA, cbufB, ebuf, xbufA, xbufB,
               gparv, semA, semB)


def _sc_worker(wid, csr_hbm, compat_hbm, xmod_hbm, gpar_hbm, out_hbm,
               csr_v, mbuf, outbuf, cbuf1, cbufA, cbufB, ebuf, xbufA, xbufB,
               gparv, semA, semB):
    base_seg = pl.multiple_of(wid * SPW, 64)

    pltpu.sync_copy(csr_hbm.at[pl.ds(base_seg, CSR_COPY)],
                    csr_v.at[pl.ds(0, CSR_COPY)])
    pltpu.sync_copy(gpar_hbm, gparv)

    r0 = _sget(csr_v, 0)
    r1 = _sget(csr_v, SPW)
    a0 = (r0 // 128) * 128
    lanes = lax.iota(jnp.int32, 16)
    mask8 = lanes < 8
    neg = jnp.float32(-jnp.inf)
    minf = jnp.full((16,), neg, jnp.float32)
    zero16 = jnp.zeros((16,), jnp.float32)
    zacc = (zero16,) * 8

    def binsearch(r):
        def step(_, lohi):
            lo, hi = lohi
            mid = lax.shift_right_logical(lo + hi, 1)
            go = _sget(csr_v, mid) <= r
            return (jnp.where(go, mid, lo), jnp.where(go, hi, mid))

        lo, _ = lax.fori_loop(0, 9, step, (jnp.int32(0), jnp.int32(SPW)))
        return lo

    nch1 = (r1 - a0 + (CR1 - 1)) // CR1

    def chunk1(k, carry):
        cursor, m_carry = carry
        off = a0 + k * CR1
        offc = pl.multiple_of(jnp.minimum(off, V - CR1), 128)
        sh = off - offc
        pltpu.sync_copy(compat_hbm.at[:, pl.ds(offc, CR1)], cbuf1)
        _fill_ebuf(ebuf, cbuf1, CR1, lanes)
        nr = jnp.minimum(CR1, r1 - off)
        end = off + nr
        seg_hi = binsearch(end - 1)

        def piece(s2, m_cur):
            a = _sget(csr_v, s2)
            b = _sget(csr_v, s2 + 1)
            a2 = jnp.maximum(a, off)
            b2 = jnp.minimum(b, end)
            m_cur = jnp.where(a >= off, minf, m_cur)

            def rowloop(j, mc):
                cvec = ebuf[pl.ds((j + sh) * 8, 16)]
                return jnp.maximum(mc, jnp.where(mask8, cvec, neg))

            m_cur = lax.fori_loop(a2 - off, b2 - off, rowloop, m_cur)

            @pl.when(b <= end)
            def _():
                mbuf[pl.ds(s2 * 16, 16)] = m_cur

            return m_cur

        m_carry = lax.fori_loop(cursor, seg_hi + 1, piece, m_carry)
        cursor = jnp.where(_sget(csr_v, seg_hi + 1) <= end, seg_hi + 1, seg_hi)
        return (cursor, m_carry)

    _ = lax.fori_loop(0, nch1, chunk1, (jnp.int32(0), minf))

    gw = gparv[pl.ds(0, 16)]
    gb = gparv[pl.ds(16, 16)]

    def finalize(sg, denom, accs, m_c):
        n = _sget(csr_v, sg + 1) - _sget(csr_v, sg)
        m0 = jnp.where(mask8, m_c, 0.0)
        z = jnp.maximum(m0 * gw + gb, 0.0)
        ez = jnp.exp(z + z)
        gate = 1.0 - 2.0 / (ez + 1.0)
        scale = gate / (denom + 1e-12)
        scale = jnp.where(n > 0, scale, 0.0)
        for v in range(8):
            outbuf[sg, pl.ds(v * 16, 16)] = accs[v] * _splat(scale, v)

    nch2 = (r1 - a0 + (CR2 - 1)) // CR2
    bufs = ((cbufA, xbufA, semA), (cbufB, xbufB, semB))

    def chunk_off(c):
        off = a0 + c * CR2
        offc = pl.multiple_of(jnp.minimum(off, V - CR2), 128)
        return off, offc

    def start_dma(c, cbuf, xbuf, sem):
        _, offc = chunk_off(c)
        pltpu.async_copy(compat_hbm.at[:, pl.ds(offc, CR2)], cbuf, sem)
        pltpu.async_copy(xmod_hbm.at[pl.ds(offc, CR2), :], xbuf, sem)

    def wait_dma(c, cbuf, xbuf, sem):
        _, offc = chunk_off(c)
        pltpu.make_async_copy(compat_hbm.at[:, pl.ds(offc, CR2)], cbuf,
                              sem).wait()
        pltpu.make_async_copy(xmod_hbm.at[pl.ds(offc, CR2), :], xbuf,
                              sem).wait()

    @pl.when(nch2 > 0)
    def _():
        start_dma(jnp.int32(0), *bufs[0])

    def chunk2(k, carry, p, cbuf2, xbuf, sem):
        cursor, denom_carry, accs_carry = carry
        guard = k < nch2
        off, offc = chunk_off(k)
        sh = off - offc

        @pl.when(guard)
        def _():
            wait_dma(k, cbuf2, xbuf, sem)

        @pl.when(k + 1 < nch2)
        def _():
            start_dma(k + 1, *bufs[1 - p])

        @pl.when(guard)
        def _():
            _fill_ebuf(ebuf, cbuf2, CR2, lanes)

        nr = jnp.where(guard, jnp.minimum(CR2, r1 - off), 0)
        end = off + nr
        seg_hi = jnp.where(guard, binsearch(end - 1), cursor - 1)

        def piece(s2, c2):
            denom, accs = c2
            a = _sget(csr_v, s2)
            b = _sget(csr_v, s2 + 1)
            a2 = jnp.maximum(a, off)
            b2 = jnp.minimum(b, end)
            fresh = a >= off
            denom = jnp.where(fresh, 0.0, denom)
            accs = tuple(jnp.where(fresh, 0.0, ac) for ac in accs)
            m_c = mbuf[pl.ds(s2 * 16, 16)]
            isn = _rsqrt_scalar(jnp.maximum(b - a, 1).astype(jnp.float32))

            def rowloop(j, c3):
                dn, ac = c3
                jb = j + sh
                cvec = ebuf[pl.ds(jb * 8, 16)]
                e = jnp.exp((cvec - m_c) * isn)
                dn = dn + jnp.where(mask8, e, 0.0)
                ac = tuple(
                    ac[v] + _splat(e, v) * xbuf[jb, pl.ds(v * 16, 16)]
                    for v in range(8))
                return (dn, ac)

            denom, accs = lax.fori_loop(a2 - off, b2 - off, rowloop,
                                        (denom, accs))

            @pl.when(b <= end)
            def _():
                finalize(s2, denom, accs, m_c)

            return (denom, accs)

        denom_carry, accs_carry = lax.fori_loop(cursor, seg_hi + 1, piece,
                                                (denom_carry, accs_carry))
        cursor = jnp.where(_sget(csr_v, seg_hi + 1) <= end, seg_hi + 1, seg_hi)
        return (cursor, denom_carry, accs_carry)

    def chunk_pair(k2, carry):
        for p in range(2):
            cbuf_p, xbuf_p, sem_p = bufs[p]
            carry = chunk2(k2 * 2 + p, carry, p, cbuf_p, xbuf_p, sem_p)
        return carry

    cursorf, _, _ = lax.fori_loop(0, (nch2 + 1) // 2, chunk_pair,
                                  (jnp.int32(0), zero16, zacc))

    def tailseg(s2, _):
        finalize(s2, zero16, zacc, zero16)
        return 0

    _ = lax.fori_loop(cursorf, SPW, tailseg, 0)

    pltpu.sync_copy(outbuf, out_hbm.at[pl.ds(base_seg, SPW), :])


def _build_sc_pool(interpret=False):
    mesh = plsc.VectorSubcoreMesh(core_axis_name="c", subcore_axis_name="s")
    return pl.kernel(
        _sc_body,
        out_type=jax.ShapeDtypeStruct((NPAD, F_MOD), jnp.float32),
        mesh=mesh,
        compiler_params=pltpu.CompilerParams(needs_layout_passes=False,
                                             use_tc_tiling_on_sc=True),
        scratch_types=[
            pltpu.VMEM((CSR_BUF,), jnp.int32),
            pltpu.VMEM((SPW * 16,), jnp.float32),
            pltpu.VMEM((SPW, F_MOD), jnp.float32),
            pltpu.VMEM((8, CR1), jnp.float32),
            pltpu.VMEM((8, CR2), jnp.float32),
            pltpu.VMEM((8, CR2), jnp.float32),
            pltpu.VMEM((max(CR1, CR2) * 8 + 16,), jnp.float32),
            pltpu.VMEM((CR2, F_MOD), jnp.float32),
            pltpu.VMEM((CR2, F_MOD), jnp.float32),
            pltpu.VMEM((32,), jnp.float32),
            pltpu.SemaphoreType.DMA,
            pltpu.SemaphoreType.DMA,
        ],
        interpret=interpret,
    )


_sc_pool = _build_sc_pool()


def kernel(x_main, x_mod, x_proj, csr_idx, W1, b1, bn_gamma, bn_beta,
           W2, b2, gate_w, gate_b):
    del x_main
    del b1
    w1p = jnp.zeros((8, F_PROJ), jnp.float32).at[:4].set(W1)
    ht, stats = _mlp_h_stats(x_proj.T, w1p)
    mu = stats[:, 0] / V
    var = stats[:, 1] / V - mu * mu
    gam = jnp.zeros((8,), jnp.float32).at[:4].set(bn_gamma)
    bet = jnp.zeros((8,), jnp.float32).at[:4].set(bn_beta)
    a = gam * lax.rsqrt(var + EPS_BN)
    bb = bet - mu * a
    w2p = jnp.zeros((8, 8), jnp.float32).at[:, :4].set(W2)
    tile128 = lambda v: jnp.tile(v.reshape(8, 1), (1, 128))
    b2p = jnp.zeros((8,), jnp.float32).at[:G].set(b2)
    compatT = _mlp_compat(ht, tile128(a), tile128(bb), w2p, tile128(b2p))

    csr32 = csr_idx.astype(jnp.int32)
    csr_pad = jnp.concatenate(
        [csr32, jnp.full(((NW - 1) * SPW + CSR_COPY - (N + 1),), V, jnp.int32)])
    gpar = jnp.concatenate([
        jnp.tile(gate_w.reshape(-1), 2).astype(jnp.float32),
        jnp.tile(gate_b.reshape(-1), 2).astype(jnp.float32),
    ])
    out = _sc_pool(csr_pad, compatT, x_mod, gpar)
    x_pool = out[:N]
    x_seen = csr_idx[1:] > csr_idx[:-1]
    return x_pool, x_seen
